# baseline (speedup 1.0000x reference)
import functools

import jax
import jax.numpy as jnp
from jax import lax
from jax.experimental import pallas as pl
from jax.experimental.pallas import tpu as pltpu
from jax.experimental.pallas import tpu_sc as plsc

_NBUF = 3
_UNROLL = 8


def kernel(spikes, embed):
    b, t, c, h = spikes.shape
    v_rows, d = embed.shape
    n = b * t * c * h
    idx = spikes.reshape(n)

    info = plsc.get_sparse_core_info()
    nw = info.num_cores * info.num_subcores
    lanes = info.num_lanes
    per_w = n // nw
    chunk = 512
    n_chunks = per_w // chunk

    mesh = plsc.VectorSubcoreMesh(core_axis_name="c", subcore_axis_name="s")

    @functools.partial(
        pl.kernel,
        mesh=mesh,
        out_type=jax.ShapeDtypeStruct((n * d,), jnp.float32),
        scratch_types=[
            pltpu.VMEM((v_rows * d,), jnp.float32),
            pltpu.VMEM((v_rows * v_rows * 2 * d,), jnp.float32),
            pltpu.VMEM((_NBUF * chunk,), jnp.int32),
            pltpu.VMEM((_NBUF * chunk * d,), jnp.float32),
            pltpu.SemaphoreType.DMA((_NBUF,)),
            pltpu.SemaphoreType.DMA((_NBUF,)),
        ],
    )
    def run(idx_hbm, table_hbm, out_hbm, table_v, ptable_v, idx_v, rows_v, isem, osem):
        wid = lax.axis_index("s") * info.num_cores + lax.axis_index("c")
        base = wid * per_w

        pltpu.sync_copy(table_hbm, table_v)

        @plsc.parallel_loop(0, v_rows * v_rows, 1, unroll=2)
        def pbuild(p):
            a = (p // v_rows) * d
            bq = (p % v_rows) * d
            o = p * (2 * d)
            for hh in range(d // lanes):
                ptable_v[pl.ds(o + hh * lanes, lanes)] = (
                    table_v[pl.ds(a + hh * lanes, lanes)])
                ptable_v[pl.ds(o + d + hh * lanes, lanes)] = (
                    table_v[pl.ds(bq + hh * lanes, lanes)])

        def idx_copy(i, s):
            return pltpu.make_async_copy(
                idx_hbm.at[pl.ds(base + i * chunk, chunk)],
                idx_v.at[pl.ds(s * chunk, chunk)], isem.at[s])

        def out_copy(i, s):
            return pltpu.make_async_copy(
                rows_v.at[pl.ds(s * chunk * d, chunk * d)],
                out_hbm.at[pl.ds((base + i * chunk) * d, chunk * d)],
                osem.at[s])

        half_c = chunk // 2

        def compute(s):
            @plsc.parallel_loop(0, half_c // lanes, 1, unroll=2)
            def kbody(k):
                iv_a = idx_v[pl.ds(s * chunk + k * lanes, lanes)]
                iv_b = idx_v[pl.ds(s * chunk + half_c + k * lanes, lanes)]
                pvec = (iv_a * v_rows + iv_b) * (2 * d)
                for u in range(lanes):
                    po = pvec[u]
                    oa = (s * chunk + k * lanes + u) * d
                    ob = (s * chunk + half_c + k * lanes + u) * d
                    rows_v[pl.ds(oa, lanes)] = ptable_v[pl.ds(po, lanes)]
                    rows_v[pl.ds(oa + lanes, lanes)] = (
                        ptable_v[pl.ds(po + lanes, lanes)])
                    rows_v[pl.ds(ob, lanes)] = (
                        ptable_v[pl.ds(po + d, lanes)])
                    rows_v[pl.ds(ob + lanes, lanes)] = (
                        ptable_v[pl.ds(po + d + lanes, lanes)])

        idx_copy(0, 0).start()

        def chunk_body(i, carry):
            s = lax.rem(i, _NBUF)

            @pl.when(i + 1 < n_chunks)
            def _():
                idx_copy(i + 1, lax.rem(i + 1, _NBUF)).start()

            idx_copy(i, s).wait()

            @pl.when(i >= _NBUF)
            def _():
                out_copy(i - _NBUF, s).wait()

            compute(s)
            out_copy(i, s).start()
            return carry

        lax.fori_loop(0, n_chunks, chunk_body, 0)
        for j in range(_NBUF):
            i = n_chunks - _NBUF + j
            out_copy(i, i % _NBUF).wait()

    out = run(idx, embed.reshape(v_rows * d))
    return out.reshape(b, t, c * h * d)


# speedup vs baseline: 1.0622x; 1.0622x over previous
import functools

import jax
import jax.numpy as jnp
from jax import lax
from jax.experimental import pallas as pl
from jax.experimental.pallas import tpu as pltpu
from jax.experimental.pallas import tpu_sc as plsc

_NBUF = 3
_UNROLL = 8


def kernel(spikes, embed):
    b, t, c, h = spikes.shape
    v_rows, d = embed.shape
    n = b * t * c * h
    idx = spikes.reshape(n)

    info = plsc.get_sparse_core_info()
    nw = info.num_cores * info.num_subcores
    lanes = info.num_lanes
    per_w = n // nw
    chunk = 1024
    n_chunks = per_w // chunk

    mesh = plsc.VectorSubcoreMesh(core_axis_name="c", subcore_axis_name="s")

    @functools.partial(
        pl.kernel,
        mesh=mesh,
        out_type=jax.ShapeDtypeStruct((n * d,), jnp.float32),
        scratch_types=[
            pltpu.VMEM((v_rows * d,), jnp.float32),
            pltpu.VMEM((_NBUF * chunk,), jnp.int32),
            pltpu.VMEM((_NBUF * chunk * d,), jnp.float32),
            pltpu.SemaphoreType.DMA((_NBUF,)),
            pltpu.SemaphoreType.DMA((_NBUF,)),
        ],
    )
    def run(idx_hbm, table_hbm, out_hbm, table_v, idx_v, rows_v, isem, osem):
        wid = lax.axis_index("s") * info.num_cores + lax.axis_index("c")
        base = wid * per_w

        pltpu.sync_copy(table_hbm, table_v)

        def idx_copy(i, s):
            return pltpu.make_async_copy(
                idx_hbm.at[pl.ds(base + i * chunk, chunk)],
                idx_v.at[pl.ds(s * chunk, chunk)], isem.at[s])

        def out_copy(i, s):
            return pltpu.make_async_copy(
                rows_v.at[pl.ds(s * chunk * d, chunk * d)],
                out_hbm.at[pl.ds((base + i * chunk) * d, chunk * d)],
                osem.at[s])

        def compute(s):
            @plsc.parallel_loop(0, chunk // lanes, 1, unroll=8)
            def kbody(k):
                ivec = idx_v[pl.ds((s * chunk + k * lanes), lanes)]
                for u in range(lanes):
                    off = ivec[u] * d
                    obase = (s * chunk + k * lanes + u) * d
                    for half in range(d // lanes):
                        rows_v[pl.ds(obase + half * lanes, lanes)] = (
                            table_v[pl.ds(off + half * lanes, lanes)])

        idx_copy(0, 0).start()

        def chunk_body(i, carry):
            s = lax.rem(i, _NBUF)

            @pl.when(i + 1 < n_chunks)
            def _():
                idx_copy(i + 1, lax.rem(i + 1, _NBUF)).start()

            idx_copy(i, s).wait()

            @pl.when(i >= _NBUF)
            def _():
                out_copy(i - _NBUF, s).wait()

            compute(s)
            out_copy(i, s).start()
            return carry

        lax.fori_loop(0, n_chunks, chunk_body, 0)
        for j in range(_NBUF):
            i = n_chunks - _NBUF + j
            out_copy(i, i % _NBUF).wait()

    out = run(idx, embed.reshape(v_rows * d))
    return out.reshape(b, t, c * h * d)


# speedup vs baseline: 1.0747x; 1.0118x over previous
import functools

import jax
import jax.numpy as jnp
from jax import lax
from jax.experimental import pallas as pl
from jax.experimental.pallas import tpu as pltpu
from jax.experimental.pallas import tpu_sc as plsc

_NBUF = 3
_UNROLL = 8


def kernel(spikes, embed):
    b, t, c, h = spikes.shape
    v_rows, d = embed.shape
    n = b * t * c * h
    idx = spikes.reshape(n)

    info = plsc.get_sparse_core_info()
    nw = info.num_cores * info.num_subcores
    lanes = info.num_lanes
    per_w = n // nw
    chunk = 1024
    n_chunks = per_w // chunk

    mesh = plsc.VectorSubcoreMesh(core_axis_name="c", subcore_axis_name="s")

    @functools.partial(
        pl.kernel,
        mesh=mesh,
        out_type=jax.ShapeDtypeStruct((n * d,), jnp.float32),
        scratch_types=[
            pltpu.VMEM((v_rows * d,), jnp.float32),
            pltpu.VMEM((_NBUF * chunk,), jnp.int32),
            pltpu.VMEM((_NBUF * chunk * d,), jnp.float32),
            pltpu.SemaphoreType.DMA((_NBUF,)),
            pltpu.SemaphoreType.DMA((_NBUF,)),
        ],
    )
    def run(idx_hbm, table_hbm, out_hbm, table_v, idx_v, rows_v, isem, osem):
        wid = lax.axis_index("s") * info.num_cores + lax.axis_index("c")
        base = wid * per_w

        pltpu.sync_copy(table_hbm, table_v)

        def idx_copy(i, s):
            return pltpu.make_async_copy(
                idx_hbm.at[pl.ds(base + i * chunk, chunk)],
                idx_v.at[pl.ds(s * chunk, chunk)], isem.at[s])

        def out_copy(i, s):
            return pltpu.make_async_copy(
                rows_v.at[pl.ds(s * chunk * d, chunk * d)],
                out_hbm.at[pl.ds((base + i * chunk) * d, chunk * d)],
                osem.at[s])

        def compute(s):
            @plsc.parallel_loop(0, chunk // lanes, 1, unroll=4)
            def kbody(k):
                ivec = idx_v[pl.ds((s * chunk + k * lanes), lanes)]
                for u in range(lanes):
                    off = ivec[u] * d
                    obase = (s * chunk + k * lanes + u) * d
                    for half in range(d // lanes):
                        rows_v[pl.ds(obase + half * lanes, lanes)] = (
                            table_v[pl.ds(off + half * lanes, lanes)])

        idx_copy(0, 0).start()

        def chunk_body(i, carry):
            s = lax.rem(i, _NBUF)

            @pl.when(i + 1 < n_chunks)
            def _():
                idx_copy(i + 1, lax.rem(i + 1, _NBUF)).start()

            idx_copy(i, s).wait()

            @pl.when(i >= _NBUF)
            def _():
                out_copy(i - _NBUF, s).wait()

            compute(s)
            out_copy(i, s).start()
            return carry

        lax.fori_loop(0, n_chunks, chunk_body, 0)
        for j in range(_NBUF):
            i = n_chunks - _NBUF + j
            out_copy(i, i % _NBUF).wait()

    out = run(idx, embed.reshape(v_rows * d))
    return out.reshape(b, t, c * h * d)
